# Initial kernel scaffold; baseline (speedup 1.0000x reference)
import jax
import jax.numpy as jnp
from jax.experimental import pallas as pl


def kernel(adjacency_list, node_features, W, b, rel_W):
    raise NotImplementedError("write your pallas kernel here")


# speedup vs baseline: 2.8475x; 2.8475x over previous
import functools

import jax
import jax.numpy as jnp
from jax import lax
from jax.experimental import pallas as pl
from jax.experimental.pallas import tpu as pltpu
from jax.experimental.pallas import tpu_sc as plsc

_N = 50000
_F = 128
_R = 8

_BN = 400

_C = 96
_FULL = _N // _C
_TAIL = _N - _FULL * _C
_NCHUNK = _FULL + 1
_NW = 32
_ITER = -(-_NCHUNK // _NW)
_ADJ_PAD = (_FULL + 2) * _C


def _tc_body(nf_ref, w_ref, b_ref, x0_ref, p_ref):
    y = jnp.dot(nf_ref[...], w_ref[...], preferred_element_type=jnp.float32)
    x0_ref[...] = y[:, :_F] + b_ref[...]
    for r in range(_R):
        p_ref[r] = y[:, _F * (r + 1):_F * (r + 2)]


def _tc_project(nf, wcat, b2):
    return pl.pallas_call(
        _tc_body,
        grid=(_N // _BN,),
        in_specs=[
            pl.BlockSpec((_BN, _F), lambda i: (i, 0)),
            pl.BlockSpec((_F, (_R + 1) * _F), lambda i: (0, 0)),
            pl.BlockSpec((1, _F), lambda i: (0, 0)),
        ],
        out_specs=[
            pl.BlockSpec((_BN, _F), lambda i: (i, 0)),
            pl.BlockSpec((_R, _BN, _F), lambda i: (0, i, 0)),
        ],
        out_shape=[
            jax.ShapeDtypeStruct((_N, _F), jnp.float32),
            jax.ShapeDtypeStruct((_R, _N, _F), jnp.float32),
        ],
    )(nf, wcat, b2)


@functools.cache
def _sc_accum_fn():
    @functools.partial(
        pl.kernel,
        out_type=jax.ShapeDtypeStruct((_N, _F), jnp.float32),
        mesh=plsc.VectorSubcoreMesh(core_axis_name="c", subcore_axis_name="s"),
        scratch_types=[
            pltpu.VMEM((_R, _C), jnp.int32),
            pltpu.VMEM((_R, _C, _F), jnp.float32),
            pltpu.VMEM((_C, _F), jnp.float32),
            pltpu.SemaphoreType.DMA,
        ],
    )
    def _sc_accum(p_hbm, x0_hbm, adj_hbm, out_hbm, idx_v, rows_v, acc_v, sem):
        wid = lax.axis_index("s") * 2 + lax.axis_index("c")

        def chunk_body(i, carry):
            cid = i * _NW + wid
            base = pl.multiple_of(cid * _C, 8)

            @pl.when(cid < _NCHUNK)
            def _():
                for r in range(_R):
                    pltpu.sync_copy(
                        adj_hbm.at[pl.ds(r * _ADJ_PAD + base, _C)], idx_v.at[r]
                    )
                copies = [
                    pltpu.async_copy(p_hbm.at[idx_v.at[r]], rows_v.at[r], sem)
                    for r in range(_R)
                ]

                @pl.when(cid < _FULL)
                def _():
                    pltpu.sync_copy(x0_hbm.at[pl.ds(base, _C)], acc_v)

                @pl.when(cid == _FULL)
                def _():
                    pltpu.sync_copy(
                        x0_hbm.at[pl.ds(base, _TAIL)], acc_v.at[pl.ds(0, _TAIL)]
                    )

                for cp in copies:
                    cp.wait()

                def row_body(row, c_):
                    for c in range(_F // 16):
                        v = acc_v[row, pl.ds(c * 16, 16)]
                        for r in range(_R):
                            v = v + rows_v[r, row, pl.ds(c * 16, 16)]
                        acc_v[row, pl.ds(c * 16, 16)] = jnp.maximum(v, 0.0)
                    return c_

                lax.fori_loop(0, _C, row_body, 0)

                @pl.when(cid < _FULL)
                def _():
                    pltpu.sync_copy(acc_v, out_hbm.at[pl.ds(base, _C)])

                @pl.when(cid == _FULL)
                def _():
                    pltpu.sync_copy(
                        acc_v.at[pl.ds(0, _TAIL)], out_hbm.at[pl.ds(base, _TAIL)]
                    )

            return carry

        lax.fori_loop(0, _ITER, chunk_body, 0)

    return _sc_accum


def kernel(adjacency_list, node_features, W, b, rel_W):
    adj = adjacency_list.astype(jnp.int32)
    nf = node_features.astype(jnp.float32)
    wcat = jnp.concatenate([W[None], rel_W], axis=0)
    wcat = jnp.transpose(wcat, (2, 0, 1)).reshape(_F, (_R + 1) * _F)
    b2 = b.reshape(1, _F).astype(jnp.float32)
    x0, p = _tc_project(nf, wcat, b2)
    pflat = p.reshape(_R * _N, _F)
    adj_pad = jnp.pad(adj, ((0, 0), (0, _ADJ_PAD - _N)))
    adj_off = adj_pad + (jnp.arange(_R, dtype=jnp.int32) * _N)[:, None]
    return _sc_accum_fn()(pflat, x0, adj_off.reshape(_R * _ADJ_PAD))
